# Initial kernel scaffold; baseline (speedup 1.0000x reference)
import jax
import jax.numpy as jnp
from jax.experimental import pallas as pl


def kernel(X, W1, W2, bias, l1_w, l1_b, l2_w, l2_b):
    raise NotImplementedError("write your pallas kernel here")


# speedup vs baseline: 1.5013x; 1.5013x over previous
import functools

import jax
import jax.numpy as jnp
from jax import lax
from jax.experimental import pallas as pl
from jax.experimental.pallas import tpu as pltpu
from jax.experimental.pallas import tpu_sc as plsc

B = 16384
F = 26
V = 1000012
D = 16
H1 = 32
H2 = 32
BF = B * F

NC = 2
NS = 16
NW = NC * NS
PER_W = BF // NW
CHUNK = 128
GPO = 8
STEP_ROWS = CHUNK * GPO
STEPS = PER_W // STEP_ROWS
IDX_ROWS = PER_W // CHUNK

TC_BLK = 1024


V16 = (V + 15) // 16


def _sc_gather(xi2d, xihi2d, w2, w1tab):
  mesh = plsc.VectorSubcoreMesh(core_axis_name="c", subcore_axis_name="s")

  @functools.partial(
      pl.kernel,
      out_type=(
          jax.ShapeDtypeStruct((BF, D), jnp.float32),
          jax.ShapeDtypeStruct((BF,), jnp.float32),
      ),
      mesh=mesh,
      compiler_params=pltpu.CompilerParams(use_tc_tiling_on_sc=False,
                                           needs_layout_passes=False),
      scratch_types=[
          pltpu.VMEM((IDX_ROWS, CHUNK), jnp.int32),
          pltpu.VMEM((IDX_ROWS, CHUNK), jnp.int32),
          pltpu.VMEM((STEP_ROWS, D), jnp.float32),
          pltpu.VMEM((STEP_ROWS, D), jnp.float32),
          pltpu.VMEM((STEP_ROWS,), jnp.float32),
          pltpu.SemaphoreType.DMA,
          pltpu.SemaphoreType.DMA,
      ],
  )
  def k(xi_hbm, xihi_hbm, w2_hbm, w1t_hbm, emb_hbm, w1o_hbm,
        idx_v, idxhi_v, rows_v, w1blk_v, w1val_v, sem2, sem1):
    wid = lax.axis_index("s") * NC + lax.axis_index("c")
    pltpu.sync_copy(xi_hbm.at[pl.ds(wid * IDX_ROWS, IDX_ROWS)], idx_v)
    pltpu.sync_copy(xihi_hbm.at[pl.ds(wid * IDX_ROWS, IDX_ROWS)], idxhi_v)

    def step(o, _):
      descs = []
      for j in range(GPO):
        r = o * GPO + j
        descs.append(pltpu.async_copy(
            w2_hbm.at[idx_v.at[r]], rows_v.at[pl.ds(j * CHUNK, CHUNK)], sem2))
        descs.append(pltpu.async_copy(
            w1t_hbm.at[idxhi_v.at[r]], w1blk_v.at[pl.ds(j * CHUNK, CHUNK)],
            sem1))
      for d in descs:
        d.wait()
      for j in range(GPO):
        r = o * GPO + j
        for g in range(CHUNK // 16):
          col = idx_v[r, pl.ds(g * 16, 16)] & 15
          row = lax.iota(jnp.int32, 16) + (j * CHUNK + g * 16)
          w1val_v[pl.ds(j * CHUNK + g * 16, 16)] = plsc.load_gather(
              w1blk_v, [row, col])
      gbase = wid * PER_W + o * STEP_ROWS
      pltpu.sync_copy(rows_v, emb_hbm.at[pl.ds(gbase, STEP_ROWS)])
      pltpu.sync_copy(w1val_v, w1o_hbm.at[pl.ds(gbase, STEP_ROWS)])
      return 0

    lax.fori_loop(0, STEPS, step, 0)

  return k(xi2d, xihi2d, w2, w1tab)


def _tc_body(emb_ref, w1_ref, l1w_ref, l1b_ref, l2w_ref, l2b_ref, s_ref,
             bias_ref, out_ref):
  e = emb_ref[...]
  h1 = jnp.dot(e, l1w_ref[...], preferred_element_type=jnp.float32)
  h1 = jnp.maximum(h1 + l1b_ref[...], 0.0)
  h2 = jnp.dot(h1, l2w_ref[...], preferred_element_type=jnp.float32)
  h2 = jnp.maximum(h2 + l2b_ref[...], 0.0)
  fm_sum = jnp.dot(e, s_ref[...], preferred_element_type=jnp.float32)
  second = 0.5 * (jnp.sum(fm_sum * fm_sum, axis=1) - jnp.sum(e * e, axis=1))
  first = jnp.sum(w1_ref[...], axis=1)
  out_ref[...] = first + second + jnp.sum(h2, axis=1) + bias_ref[0]


def _tc_dense(emb, w1g, l1_w, l1_b, l2_w, l2_b, bias):
  smat = jnp.tile(jnp.eye(D, dtype=jnp.float32), (F, 1))
  grid = B // TC_BLK
  return pl.pallas_call(
      _tc_body,
      grid=(grid,),
      in_specs=[
          pl.BlockSpec((TC_BLK, F * D), lambda i: (i, 0)),
          pl.BlockSpec((TC_BLK, F), lambda i: (i, 0)),
          pl.BlockSpec((F * D, H1), lambda i: (0, 0)),
          pl.BlockSpec((1, H1), lambda i: (0, 0)),
          pl.BlockSpec((H1, H2), lambda i: (0, 0)),
          pl.BlockSpec((1, H2), lambda i: (0, 0)),
          pl.BlockSpec((F * D, D), lambda i: (0, 0)),
          pl.BlockSpec(memory_space=pltpu.SMEM),
      ],
      out_specs=pl.BlockSpec((TC_BLK,), lambda i: (i,)),
      out_shape=jax.ShapeDtypeStruct((B,), jnp.float32),
  )(emb, w1g, l1_w, l1_b.reshape(1, H1), l2_w, l2_b.reshape(1, H2), smat,
    bias)


def kernel(X, W1, W2, bias, l1_w, l1_b, l2_w, l2_b):
  xi = X.reshape(BF).astype(jnp.int32)
  xi2d = xi.reshape(BF // CHUNK, CHUNK)
  xihi2d = (xi >> 4).reshape(BF // CHUNK, CHUNK)
  w1tab = jnp.pad(W1.reshape(-1), (0, V16 * 16 - V)).reshape(V16, D)
  emb, w1g = _sc_gather(xi2d, xihi2d, W2, w1tab)
  return _tc_dense(emb.reshape(B, F * D), w1g.reshape(B, F),
                   l1_w, l1_b, l2_w, l2_b, bias)
